# Initial kernel scaffold; baseline (speedup 1.0000x reference)
import jax
import jax.numpy as jnp
from jax.experimental import pallas as pl


def kernel(pred, target):
    raise NotImplementedError("write your pallas kernel here")


# speedup vs baseline: 1.5535x; 1.5535x over previous
import functools
import math

import jax
import jax.numpy as jnp
from jax.experimental import pallas as pl
from jax.experimental.pallas import tpu as pltpu

_SMOOTHING = 0.1


def _block_kernel(pred_ref, tgt_ref, out_ref, *, a_coef, d_coef):
    x = pred_ref[0]
    t = tgt_ref[0]
    m = jnp.max(x, axis=0)
    s = jnp.sum(jnp.exp(x - m[None]), axis=0)
    lse = m + jnp.log(s)
    cls = jax.lax.broadcasted_iota(jnp.int32, x.shape, 0)
    ptgt = jnp.sum(jnp.where(cls == t[None], x, 0.0), axis=0)
    partial = (a_coef * jnp.sum(lse)
               - _SMOOTHING * jnp.sum(x)
               - d_coef * jnp.sum(ptgt))
    out_ref[...] = jnp.reshape(partial, (1, 1, 1))


def kernel(pred, target):
    n, c, h, w = pred.shape
    conf = 1.0 - _SMOOTHING
    tgt_val = conf + _SMOOTHING / c
    a_coef = _SMOOTHING * c + (tgt_val - _SMOOTHING)
    d_coef = tgt_val - _SMOOTHING
    k_const = tgt_val * math.log(tgt_val) + (c - 1) * _SMOOTHING * math.log(_SMOOTHING)

    bh = 64
    hb = h // bh
    grid = (n * hb,)

    partials = pl.pallas_call(
        functools.partial(_block_kernel, a_coef=a_coef, d_coef=d_coef),
        grid=grid,
        in_specs=[
            pl.BlockSpec((1, c, bh, w), lambda i: (i // hb, 0, i % hb, 0)),
            pl.BlockSpec((1, bh, w), lambda i: (i // hb, i % hb, 0)),
        ],
        out_specs=pl.BlockSpec((1, 1, 1), lambda i: (i, 0, 0)),
        out_shape=jax.ShapeDtypeStruct((grid[0], 1, 1), jnp.float32),
        compiler_params=pltpu.CompilerParams(
            dimension_semantics=("arbitrary",),
        ),
    )(pred, target)

    pixels = n * h * w
    total = jnp.sum(partials) + pixels * k_const
    return total / (n * c * h * w)


# speedup vs baseline: 1.6701x; 1.0751x over previous
import functools
import math

import jax
import jax.numpy as jnp
from jax.experimental import pallas as pl
from jax.experimental.pallas import tpu as pltpu

_SMOOTHING = 0.1


def _block_kernel(pred_ref, tgt_ref, out_ref, *, a_coef, d_coef):
    x = pred_ref[0]
    t = tgt_ref[0]
    m = jnp.max(x, axis=0)
    s = jnp.sum(jnp.exp(x - m[None]), axis=0)
    lse = m + jnp.log(s)
    cls = jax.lax.broadcasted_iota(jnp.int32, x.shape, 0)
    ptgt = jnp.sum(jnp.where(cls == t[None], x, 0.0), axis=0)
    partial = (a_coef * jnp.sum(lse)
               - _SMOOTHING * jnp.sum(x)
               - d_coef * jnp.sum(ptgt))
    out_ref[...] = jnp.reshape(partial, (1, 1, 1))


def kernel(pred, target):
    n, c, h, w = pred.shape
    conf = 1.0 - _SMOOTHING
    tgt_val = conf + _SMOOTHING / c
    a_coef = _SMOOTHING * c + (tgt_val - _SMOOTHING)
    d_coef = tgt_val - _SMOOTHING
    k_const = tgt_val * math.log(tgt_val) + (c - 1) * _SMOOTHING * math.log(_SMOOTHING)

    bh = 64
    hb = h // bh
    grid = (n * hb,)

    partials = pl.pallas_call(
        functools.partial(_block_kernel, a_coef=a_coef, d_coef=d_coef),
        grid=grid,
        in_specs=[
            pl.BlockSpec((1, c, bh, w), lambda i: (i // hb, 0, i % hb, 0)),
            pl.BlockSpec((1, bh, w), lambda i: (i // hb, i % hb, 0)),
        ],
        out_specs=pl.BlockSpec((1, 1, 1), lambda i: (i, 0, 0)),
        out_shape=jax.ShapeDtypeStruct((grid[0], 1, 1), jnp.float32),
        compiler_params=pltpu.CompilerParams(
            dimension_semantics=("parallel",),
        ),
    )(pred, target)

    pixels = n * h * w
    total = jnp.sum(partials) + pixels * k_const
    return total / (n * c * h * w)
